# baseline (speedup 1.0000x reference)
import functools
from math import pi as PI

import jax
import jax.numpy as jnp
from jax import lax
from jax.experimental import pallas as pl
from jax.experimental.pallas import tpu as pltpu
from jax.experimental.pallas import tpu_sc as plsc

_N_MOLS = 128
_CONFS_PER_MOL = 4
_ATOMS = 32
_N_CONF = _N_MOLS * _CONFS_PER_MOL
_N = _N_CONF * _ATOMS
_HID = 128
_NF = 128
_NG = 50
_NI = 6
_CUTOFF = 10.0
_CB = 8
_A = _CB * _ATOMS
_P = _A * _ATOMS

_GAMMA = _CUTOFF / (_NG - 1)
_COEFF = -0.5 / _GAMMA ** 2
_LOG2 = 0.6931471805599453
_LOG2E = 1.4426950408889634


_SC_WORKERS = 32
_B_PER_W = _N // _SC_WORKERS


@functools.partial(
    pl.kernel,
    mesh=plsc.VectorSubcoreMesh(core_axis_name="c", subcore_axis_name="s"),
    out_type=jax.ShapeDtypeStruct((_N, _HID), jnp.float32),
    scratch_types=[
        pltpu.VMEM((_B_PER_W,), jnp.int32),
        pltpu.VMEM((_B_PER_W, _HID), jnp.float32),
        pltpu.SemaphoreType.DMA,
    ],
)
def _sc_embed(table_hbm, idx_hbm, out_hbm, idx_v, rows_v, sem):
    wid = lax.axis_index("s") * 2 + lax.axis_index("c")
    base = wid * _B_PER_W
    pltpu.sync_copy(idx_hbm.at[pl.ds(base, _B_PER_W)], idx_v)
    pltpu.async_copy(table_hbm.at[idx_v], rows_v, sem).wait()
    pltpu.sync_copy(rows_v, out_hbm.at[pl.ds(base, _B_PER_W)])


def _ssp(x):
    return jnp.maximum(x, 0.0) + jnp.log1p(jnp.exp(-jnp.abs(x))) - _LOG2


def _main_body(px_ref, py_ref, pz_ref, h0_ref,
               Wg1_ref, bg1_ref, Wg2_ref, bg2_ref,
               Wc1_ref, Wc2_ref, bc2_ref, Wl_ref, bl_ref,
               W1_ref, b1_ref, W2_ref, b2_ref,
               h_out_ref, conf_out_ref):
    f32 = jnp.float32

    d2 = jnp.zeros((_CB, _ATOMS, _ATOMS), f32)
    for c_ref in (px_ref, py_ref, pz_ref):
        cv = c_ref[...]
        diff = cv[:, :, None] - cv[:, None, :]
        d2 = d2 + diff * diff
    ew = jnp.sqrt(d2)

    iota_d = lax.broadcasted_iota(jnp.int32, (_CB, _ATOMS, _ATOMS), 1)
    iota_s = lax.broadcasted_iota(jnp.int32, (_CB, _ATOMS, _ATOMS), 2)
    mask = (iota_d != iota_s) & (ew < _CUTOFF)
    C = jnp.where(mask, 0.5 * (jnp.cos(ew * (PI / _CUTOFF)) + 1.0), 0.0)
    C4 = C[:, :, :, None]

    gi = lax.broadcasted_iota(jnp.int32, (_CB, _ATOMS, _ATOMS, _NG), 3)
    off = gi.astype(f32) * _GAMMA
    delta = ew[:, :, :, None] - off
    ea = jnp.exp(_COEFF * delta * delta).reshape(_P, _NG)

    h = h0_ref[...]

    bg1 = bg1_ref[...]
    bg2 = bg2_ref[...]
    bc2 = bc2_ref[...]
    bl = bl_ref[...]

    for i in range(_NI):
        x = jnp.dot(ea, Wg1_ref[i], preferred_element_type=f32) + bg1[i:i + 1, :]
        g = jnp.log(jnp.exp2(x * _LOG2E) + 1.0) - _LOG2
        wf = (jnp.dot(g, Wg2_ref[i], preferred_element_type=f32)
              + bg2[i:i + 1, :])
        wf4 = wf.reshape(_CB, _ATOMS, _ATOMS, _NF) * C4
        xf = jnp.dot(h, Wc1_ref[i], preferred_element_type=f32)
        xf4 = xf.reshape(_CB, 1, _ATOMS, _NF)
        agg = jnp.sum(wf4 * xf4, axis=2).reshape(_A, _NF)
        t = _ssp(jnp.dot(agg, Wc2_ref[i], preferred_element_type=f32)
                 + bc2[i:i + 1, :])
        h = h + jnp.dot(t, Wl_ref[i], preferred_element_type=f32) + bl[i:i + 1, :]

    t = _ssp(jnp.dot(h, W1_ref[...], preferred_element_type=f32) + b1_ref[...])
    hf = jnp.dot(t, W2_ref[...], preferred_element_type=f32) + b2_ref[...]
    h_out_ref[...] = hf
    conf_out_ref[...] = jnp.sum(hf.reshape(_CB, _ATOMS, _HID), axis=1)


def _readout_body(conf_ref, Wh1_ref, bh1_ref, Wh2_ref, bh2_ref,
                  mol_ref, pred_ref):
    f32 = jnp.float32
    conf = conf_ref[...]
    mol = jnp.sum(conf.reshape(_N_MOLS, _CONFS_PER_MOL, _HID), axis=1)
    mol_ref[...] = mol
    t = _ssp(jnp.dot(mol, Wh1_ref[...], preferred_element_type=f32)
             + bh1_ref[...])
    pred_ref[...] = (jnp.dot(t, Wh2_ref[...], preferred_element_type=f32)
                     + bh2_ref[...])


@functools.partial(jax.jit, static_argnames=())
def kernel(pos, z, edge_index, atom_to_conf, conf_to_mol, num_atoms_per_mol,
           num_confs_per_mol, emb_table, Wg1, bg1, Wg2, bg2, Wc1, Wc2, bc2,
           Wl, bl, W1, b1, W2, b2, Wh1, bh1, Wh2, bh2):
    f32 = jnp.float32
    px = pos[:, 0].reshape(_N_CONF, _ATOMS)
    py = pos[:, 1].reshape(_N_CONF, _ATOMS)
    pz = pos[:, 2].reshape(_N_CONF, _ATOMS)
    h0 = _sc_embed(emb_table, z)

    grid = (_N_CONF // _CB,)
    full = lambda a: pl.BlockSpec(a.shape, lambda g: (0,) * a.ndim)

    h_out, conf_out = pl.pallas_call(
        _main_body,
        grid=grid,
        in_specs=[
            pl.BlockSpec((_CB, _ATOMS), lambda g: (g, 0)),
            pl.BlockSpec((_CB, _ATOMS), lambda g: (g, 0)),
            pl.BlockSpec((_CB, _ATOMS), lambda g: (g, 0)),
            pl.BlockSpec((_A, _HID), lambda g: (g, 0)),
            full(Wg1), full(bg1), full(Wg2), full(bg2),
            full(Wc1), full(Wc2), full(bc2), full(Wl), full(bl),
            full(W1), pl.BlockSpec((1, _HID), lambda g: (0, 0)),
            full(W2), pl.BlockSpec((1, _HID), lambda g: (0, 0)),
        ],
        out_specs=[
            pl.BlockSpec((_A, _HID), lambda g: (g, 0)),
            pl.BlockSpec((_CB, _HID), lambda g: (g, 0)),
        ],
        out_shape=[
            jax.ShapeDtypeStruct((_N, _HID), f32),
            jax.ShapeDtypeStruct((_N_CONF, _HID), f32),
        ],
        compiler_params=pltpu.CompilerParams(
            dimension_semantics=("arbitrary",)),
    )(px, py, pz, h0, Wg1, bg1, Wg2, bg2, Wc1, Wc2, bc2, Wl, bl,
      W1, b1.reshape(1, _HID), W2, b2.reshape(1, _HID))

    mol_emb, pred2 = pl.pallas_call(
        _readout_body,
        out_shape=[
            jax.ShapeDtypeStruct((_N_MOLS, _HID), f32),
            jax.ShapeDtypeStruct((_N_MOLS, 1), f32),
        ],
    )(conf_out, Wh1, bh1.reshape(1, _HID // 2), Wh2, bh2.reshape(1, 1))

    pred = pred2[:, 0]
    emb = h_out.reshape(_N_MOLS, _CONFS_PER_MOL, _ATOMS, _HID)
    return (pred, emb, mol_emb)


# speedup vs baseline: 1.0676x; 1.0676x over previous
import functools
from math import pi as PI

import jax
import jax.numpy as jnp
from jax import lax
from jax.experimental import pallas as pl
from jax.experimental.pallas import tpu as pltpu
from jax.experimental.pallas import tpu_sc as plsc

_N_MOLS = 128
_CONFS_PER_MOL = 4
_ATOMS = 32
_N_CONF = _N_MOLS * _CONFS_PER_MOL
_N = _N_CONF * _ATOMS
_HID = 128
_NF = 128
_NG = 50
_NI = 6
_CUTOFF = 10.0
_CB = 16
_A = _CB * _ATOMS
_P = _A * _ATOMS

_GAMMA = _CUTOFF / (_NG - 1)
_COEFF = -0.5 / _GAMMA ** 2
_LOG2 = 0.6931471805599453
_LOG2E = 1.4426950408889634


_SC_WORKERS = 32
_B_PER_W = _N // _SC_WORKERS


@functools.partial(
    pl.kernel,
    mesh=plsc.VectorSubcoreMesh(core_axis_name="c", subcore_axis_name="s"),
    out_type=jax.ShapeDtypeStruct((_N, _HID), jnp.float32),
    scratch_types=[
        pltpu.VMEM((_B_PER_W,), jnp.int32),
        pltpu.VMEM((_B_PER_W, _HID), jnp.float32),
        pltpu.SemaphoreType.DMA,
    ],
)
def _sc_embed(table_hbm, idx_hbm, out_hbm, idx_v, rows_v, sem):
    wid = lax.axis_index("s") * 2 + lax.axis_index("c")
    base = wid * _B_PER_W
    pltpu.sync_copy(idx_hbm.at[pl.ds(base, _B_PER_W)], idx_v)
    pltpu.async_copy(table_hbm.at[idx_v], rows_v, sem).wait()
    pltpu.sync_copy(rows_v, out_hbm.at[pl.ds(base, _B_PER_W)])


def _ssp(x):
    return jnp.maximum(x, 0.0) + jnp.log1p(jnp.exp(-jnp.abs(x))) - _LOG2


def _main_body(px_ref, py_ref, pz_ref, h0_ref,
               Wg1_ref, bg1_ref, Wg2_ref, bg2_ref,
               Wc1_ref, Wc2_ref, bc2_ref, Wl_ref, bl_ref,
               W1_ref, b1_ref, W2_ref, b2_ref,
               h_out_ref, conf_out_ref):
    f32 = jnp.float32

    d2 = jnp.zeros((_CB, _ATOMS, _ATOMS), f32)
    for c_ref in (px_ref, py_ref, pz_ref):
        cv = c_ref[...]
        diff = cv[:, :, None] - cv[:, None, :]
        d2 = d2 + diff * diff
    ew = jnp.sqrt(d2)

    iota_d = lax.broadcasted_iota(jnp.int32, (_CB, _ATOMS, _ATOMS), 1)
    iota_s = lax.broadcasted_iota(jnp.int32, (_CB, _ATOMS, _ATOMS), 2)
    mask = (iota_d != iota_s) & (ew < _CUTOFF)
    C = jnp.where(mask, 0.5 * (jnp.cos(ew * (PI / _CUTOFF)) + 1.0), 0.0)
    C4 = C[:, :, :, None]

    gi = lax.broadcasted_iota(jnp.int32, (_CB, _ATOMS, _ATOMS, _NG), 3)
    off = gi.astype(f32) * _GAMMA
    delta = ew[:, :, :, None] - off
    ea = jnp.exp(_COEFF * delta * delta).reshape(_P, _NG)

    h = h0_ref[...]

    bg1 = bg1_ref[...]
    bg2 = bg2_ref[...]
    bc2 = bc2_ref[...]
    bl = bl_ref[...]

    for i in range(_NI):
        x = jnp.dot(ea, Wg1_ref[i], preferred_element_type=f32) + bg1[i:i + 1, :]
        g = jnp.log(jnp.exp2(x * _LOG2E) + 1.0) - _LOG2
        wf = (jnp.dot(g, Wg2_ref[i], preferred_element_type=f32)
              + bg2[i:i + 1, :])
        wf4 = wf.reshape(_CB, _ATOMS, _ATOMS, _NF) * C4
        xf = jnp.dot(h, Wc1_ref[i], preferred_element_type=f32)
        xf4 = xf.reshape(_CB, 1, _ATOMS, _NF)
        agg = jnp.sum(wf4 * xf4, axis=2).reshape(_A, _NF)
        t = _ssp(jnp.dot(agg, Wc2_ref[i], preferred_element_type=f32)
                 + bc2[i:i + 1, :])
        h = h + jnp.dot(t, Wl_ref[i], preferred_element_type=f32) + bl[i:i + 1, :]

    t = _ssp(jnp.dot(h, W1_ref[...], preferred_element_type=f32) + b1_ref[...])
    hf = jnp.dot(t, W2_ref[...], preferred_element_type=f32) + b2_ref[...]
    h_out_ref[...] = hf
    conf_out_ref[...] = jnp.sum(hf.reshape(_CB, _ATOMS, _HID), axis=1)


def _readout_body(conf_ref, Wh1_ref, bh1_ref, Wh2_ref, bh2_ref,
                  mol_ref, pred_ref):
    f32 = jnp.float32
    conf = conf_ref[...]
    mol = jnp.sum(conf.reshape(_N_MOLS, _CONFS_PER_MOL, _HID), axis=1)
    mol_ref[...] = mol
    t = _ssp(jnp.dot(mol, Wh1_ref[...], preferred_element_type=f32)
             + bh1_ref[...])
    pred_ref[...] = (jnp.dot(t, Wh2_ref[...], preferred_element_type=f32)
                     + bh2_ref[...])


@functools.partial(jax.jit, static_argnames=())
def kernel(pos, z, edge_index, atom_to_conf, conf_to_mol, num_atoms_per_mol,
           num_confs_per_mol, emb_table, Wg1, bg1, Wg2, bg2, Wc1, Wc2, bc2,
           Wl, bl, W1, b1, W2, b2, Wh1, bh1, Wh2, bh2):
    f32 = jnp.float32
    px = pos[:, 0].reshape(_N_CONF, _ATOMS)
    py = pos[:, 1].reshape(_N_CONF, _ATOMS)
    pz = pos[:, 2].reshape(_N_CONF, _ATOMS)
    h0 = _sc_embed(emb_table, z)

    grid = (_N_CONF // _CB,)
    full = lambda a: pl.BlockSpec(a.shape, lambda g: (0,) * a.ndim)

    h_out, conf_out = pl.pallas_call(
        _main_body,
        grid=grid,
        in_specs=[
            pl.BlockSpec((_CB, _ATOMS), lambda g: (g, 0)),
            pl.BlockSpec((_CB, _ATOMS), lambda g: (g, 0)),
            pl.BlockSpec((_CB, _ATOMS), lambda g: (g, 0)),
            pl.BlockSpec((_A, _HID), lambda g: (g, 0)),
            full(Wg1), full(bg1), full(Wg2), full(bg2),
            full(Wc1), full(Wc2), full(bc2), full(Wl), full(bl),
            full(W1), pl.BlockSpec((1, _HID), lambda g: (0, 0)),
            full(W2), pl.BlockSpec((1, _HID), lambda g: (0, 0)),
        ],
        out_specs=[
            pl.BlockSpec((_A, _HID), lambda g: (g, 0)),
            pl.BlockSpec((_CB, _HID), lambda g: (g, 0)),
        ],
        out_shape=[
            jax.ShapeDtypeStruct((_N, _HID), f32),
            jax.ShapeDtypeStruct((_N_CONF, _HID), f32),
        ],
        compiler_params=pltpu.CompilerParams(
            dimension_semantics=("arbitrary",)),
    )(px, py, pz, h0, Wg1, bg1, Wg2, bg2, Wc1, Wc2, bc2, Wl, bl,
      W1, b1.reshape(1, _HID), W2, b2.reshape(1, _HID))

    mol_emb, pred2 = pl.pallas_call(
        _readout_body,
        out_shape=[
            jax.ShapeDtypeStruct((_N_MOLS, _HID), f32),
            jax.ShapeDtypeStruct((_N_MOLS, 1), f32),
        ],
    )(conf_out, Wh1, bh1.reshape(1, _HID // 2), Wh2, bh2.reshape(1, 1))

    pred = pred2[:, 0]
    emb = h_out.reshape(_N_MOLS, _CONFS_PER_MOL, _ATOMS, _HID)
    return (pred, emb, mol_emb)
